# baseline (speedup 1.0000x reference)
import jax
import jax.numpy as jnp
from jax.experimental import pallas as pl
from jax.experimental.pallas import tpu as pltpu

_N_CHUNKS = 8


def _dma_copy(x_ref, o_ref, sems):
    for i in range(_N_CHUNKS):
        pltpu.make_async_copy(x_ref.at[i], o_ref.at[i], sems.at[i]).start()
    for i in range(_N_CHUNKS):
        pltpu.make_async_copy(x_ref.at[i], o_ref.at[i], sems.at[i]).wait()


def kernel(x, s):
    del s
    b, m, n = x.shape
    xf = x.reshape(_N_CHUNKS, (b * m) // _N_CHUNKS, n)
    out = pl.pallas_call(
        _dma_copy,
        in_specs=[pl.BlockSpec(memory_space=pl.ANY)],
        out_specs=pl.BlockSpec(memory_space=pl.ANY),
        out_shape=jax.ShapeDtypeStruct(xf.shape, x.dtype),
        scratch_shapes=[pltpu.SemaphoreType.DMA((_N_CHUNKS,))],
    )(xf)
    return out.reshape(b, m, n)


# speedup vs baseline: 47.1117x; 47.1117x over previous
import jax
import jax.numpy as jnp
from jax.experimental import pallas as pl


def _copy_block(x_ref, o_ref):
    o_ref[...] = x_ref[...]


def kernel(x, s):
    del s
    b, m, n = x.shape
    xf = x.reshape(b * m, n)
    rows = b * m
    block_rows = 512
    grid = (rows // block_rows,)
    out = pl.pallas_call(
        _copy_block,
        grid=grid,
        in_specs=[pl.BlockSpec((block_rows, n), lambda i: (i, 0))],
        out_specs=pl.BlockSpec((block_rows, n), lambda i: (i, 0)),
        out_shape=jax.ShapeDtypeStruct((rows, n), x.dtype),
    )(xf)
    return out.reshape(b, m, n)


# speedup vs baseline: 48.0250x; 1.0194x over previous
import jax
import jax.numpy as jnp
from jax.experimental import pallas as pl


def _copy_block(x_ref, o_ref):
    o_ref[...] = x_ref[...]


def kernel(x, s):
    del s
    b, m, n = x.shape
    xf = x.reshape(b * m, n)
    rows = b * m
    block_rows = 1024
    grid = (rows // block_rows,)
    out = pl.pallas_call(
        _copy_block,
        grid=grid,
        in_specs=[pl.BlockSpec((block_rows, n), lambda i: (i, 0))],
        out_specs=pl.BlockSpec((block_rows, n), lambda i: (i, 0)),
        out_shape=jax.ShapeDtypeStruct((rows, n), x.dtype),
    )(xf)
    return out.reshape(b, m, n)
